# baseline (speedup 1.0000x reference)
import dataclasses
import functools

import jax
import jax.numpy as jnp
from jax import lax
from jax.experimental import pallas as pl
from jax.experimental.pallas import tpu as pltpu
from jax.experimental.pallas import tpu_sc as plsc

DIM = 64
BATCH = 16384
REG_COEF = 1e-05
NROWS = 100000
NPAIR = NROWS // 2
NC = 2
NS = 16
LANES = 16
NW = NC * NS
BPW = BATCH // NW
CHUNK = 128
NCHUNK = BPW // CHUNK
GPC = CHUNK // LANES

TBLK = 512


NTBLK = (NROWS + TBLK - 1) // TBLK
NOUT = NTBLK * (TBLK // 2)
HALF = TBLK // 2


def _pack_blk(t):
    a = jnp.transpose(t, (1, 0))
    return jnp.concatenate([a[0:HALF, :], a[HALF:TBLK, :]], axis=1)


def _tr_body(ttu_ref, tti_ref, outu_ref, outi_ref):
    outu_ref[...] = _pack_blk(ttu_ref[...])
    outi_ref[...] = _pack_blk(tti_ref[...])


def _to_rowmajor(user_table, item_table):
    return pl.pallas_call(
        _tr_body,
        grid=(NTBLK,),
        in_specs=[
            pl.BlockSpec((DIM, TBLK), lambda i: (0, i)),
            pl.BlockSpec((DIM, TBLK), lambda i: (0, i)),
        ],
        out_specs=[
            pl.BlockSpec((HALF, 128), lambda i: (i, 0)),
            pl.BlockSpec((HALF, 128), lambda i: (i, 0)),
        ],
        out_shape=[
            jax.ShapeDtypeStruct((NOUT, 128), jnp.float32),
            jax.ShapeDtypeStruct((NOUT, 128), jnp.float32),
        ],
    )(user_table.T, item_table.T)


def _sc_body(gidx_u, gidx_p, gidx_n, colb_u, colb_p, colb_n,
             utab, itab, diff_hbm, sq_hbm,
             iu_v, ip_v, in_v, cu_v, cp_v, cn_v,
             ru0, ru1, rp0, rp1, rn0, rn1,
             scores_v, sq_v, sem0, sem1):
    wid = lax.axis_index("s") * NC + lax.axis_index("c")

    pltpu.sync_copy(gidx_u.at[wid], iu_v)
    pltpu.sync_copy(gidx_p.at[wid], ip_v)
    pltpu.sync_copy(gidx_n.at[wid], in_v)
    pltpu.sync_copy(colb_u.at[wid], cu_v)
    pltpu.sync_copy(colb_p.at[wid], cp_v)
    pltpu.sync_copy(colb_n.at[wid], cn_v)

    rbufs = [(ru0, rp0, rn0), (ru1, rp1, rn1)]
    sems = [sem0, sem1]

    def fire(c):
        ru, rp, rn = rbufs[c % 2]
        sem = sems[c % 2]
        return [
            pltpu.async_copy(utab.at[iu_v.at[c]], ru, sem),
            pltpu.async_copy(itab.at[ip_v.at[c]], rp, sem),
            pltpu.async_copy(itab.at[in_v.at[c]], rn, sem),
        ]

    sq_v[...] = jnp.zeros((LANES,), jnp.float32)
    iota = lax.iota(jnp.int32, LANES)

    pending = fire(0)
    for c in range(NCHUNK):
        nxt = fire(c + 1) if c + 1 < NCHUNK else []
        for cp in pending:
            cp.wait()
        pending = nxt
        ru, rp, rn = rbufs[c % 2]

        @pl.loop(0, GPC)
        def _group(g):
            row = g * LANES + iota
            cu = cu_v[c, pl.ds(g * LANES, LANES)]
            cp_ = cp_v[c, pl.ds(g * LANES, LANES)]
            cn = cn_v[c, pl.ds(g * LANES, LANES)]
            pos = jnp.zeros((LANES,), jnp.float32)
            neg = jnp.zeros((LANES,), jnp.float32)
            sq = jnp.zeros((LANES,), jnp.float32)
            for d in range(DIM):
                u = plsc.load_gather(ru, [row, cu + d])
                p = plsc.load_gather(rp, [row, cp_ + d])
                n = plsc.load_gather(rn, [row, cn + d])
                pos = pos + u * p
                neg = neg + u * n
                sq = sq + (u * u + p * p + n * n)
            scores_v[pl.ds(c * CHUNK + g * LANES, LANES)] = pos - neg
            sq_v[...] += sq

    pltpu.sync_copy(scores_v, diff_hbm.at[pl.ds(wid * BPW, BPW)])
    pltpu.sync_copy(sq_v, sq_hbm.at[wid])


def _loss_body(diff_ref, sq_ref, out_ref):
    d = diff_ref[...]
    sp = jnp.maximum(-d, 0.0) + jnp.log1p(jnp.exp(-jnp.abs(d)))
    out_ref[0] = jnp.sum(sp) * (1.0 / BATCH)
    out_ref[1] = (0.5 * REG_COEF) * jnp.sum(sq_ref[...])


@jax.jit
def kernel(userids, itemids_pos, itemids_neg, user_table, item_table):
    uid = userids.astype(jnp.int32)
    pid = itemids_pos.astype(jnp.int32)
    nid = itemids_neg.astype(jnp.int32)
    shp = (NW, NCHUNK, CHUNK)
    gidx = [((x // TBLK) * HALF + (x % HALF)).reshape(shp)
            for x in (uid, pid, nid)]
    colb = [(((x // HALF) & 1) << 6).reshape(shp) for x in (uid, pid, nid)]

    utab, itab = _to_rowmajor(user_table, item_table)

    mesh = plsc.VectorSubcoreMesh(
        core_axis_name="c", subcore_axis_name="s",
        num_cores=NC, num_subcores=NS)

    cp = pltpu.CompilerParams()
    if "needs_layout_passes" in pltpu.CompilerParams.__dataclass_fields__:
        cp = dataclasses.replace(cp, needs_layout_passes=False)
    if "use_tc_tiling_on_sc" in pltpu.CompilerParams.__dataclass_fields__:
        cp = dataclasses.replace(cp, use_tc_tiling_on_sc=False)

    idx_t = pltpu.VMEM((NCHUNK, CHUNK), jnp.int32)
    row_t = pltpu.VMEM((CHUNK, 128), jnp.float32)
    sc = pl.kernel(
        _sc_body,
        compiler_params=cp,
        out_type=[
            jax.ShapeDtypeStruct((BATCH,), jnp.float32),
            jax.ShapeDtypeStruct((NW, LANES), jnp.float32),
        ],
        mesh=mesh,
        scratch_types=[
            idx_t, idx_t, idx_t, idx_t, idx_t, idx_t,
            row_t, row_t, row_t, row_t, row_t, row_t,
            pltpu.VMEM((BPW,), jnp.float32),
            pltpu.VMEM((LANES,), jnp.float32),
            pltpu.SemaphoreType.DMA,
            pltpu.SemaphoreType.DMA,
        ],
    )
    diff, sq = sc(*gidx, *colb, utab, itab)

    out = pl.pallas_call(
        _loss_body,
        out_shape=jax.ShapeDtypeStruct((2,), jnp.float32),
        out_specs=pl.BlockSpec(memory_space=pltpu.SMEM),
    )(diff.reshape(BATCH // 128, 128), sq)
    return out[0], out[1]
